# baseline (speedup 1.0000x reference)
import functools

import jax
import jax.numpy as jnp
from jax import lax
from jax.experimental import pallas as pl
from jax.experimental.pallas import tpu as pltpu
from jax.experimental.pallas import tpu_sc as plsc

LEVELS = 256
OFF = 5
B, H, W = 16, 512, 512
PAIRS_PER_ROW = W - OFF
NPAIRS = H * PAIRS_PER_ROW
NC, NS, L = 2, 16, 16
NW = NC * NS
ROWS_PER_WORKER = H // 2
CHUNK_ROWS = 64
NCHUNK = ROWS_PER_WORKER // CHUNK_ROWS
VECS_PER_ROW = W // L
NACC = 4
MAGIC = 12582912.0
_PERM_DN = lax.GatherDimensionNumbers(
    offset_dims=(), collapsed_slice_dims=(0,), start_index_map=(0,))


def _lane_perm(vec, idx):
    return lax.gather(vec, idx[:, None], _PERM_DN, slice_sizes=(1,),
                      mode=lax.GatherScatterMode.PROMISE_IN_BOUNDS)


def _glcm_body(x_hbm, out_hbm, buf0, buf1, stage, smin, smax, acc4, res,
               sem0, sem1):
    c = lax.axis_index("c")
    s = lax.axis_index("s")
    img = c * (B // NC) + s // 2
    half = s % 2
    row0 = img * H + half * ROWS_PER_WORKER

    def fetch(ch, buf, sem):
        return pltpu.async_copy(
            x_hbm.at[pl.ds(row0 + ch * CHUNK_ROWS, CHUNK_ROWS), :], buf, sem)

    def wait_fetch(ch, buf, sem):
        pltpu.make_async_copy(
            x_hbm.at[pl.ds(row0 + ch * CHUNK_ROWS, CHUNK_ROWS), :], buf,
            sem).wait()

    iota = lax.iota(jnp.int32, L)
    idx5 = (iota + OFF) & (L - 1)
    ge5 = iota >= OFF
    tail_mask = iota < (PAIRS_PER_ROW - (VECS_PER_ROW - 1) * L)

    NPAIR = NCHUNK // 2

    def run_pass(process, carry):
        def pair_body(p, carry):
            ch = 2 * p
            fetch(ch + 1, buf1, sem1)
            wait_fetch(ch, buf0, sem0)
            carry = lax.fori_loop(0, CHUNK_ROWS, process(buf0), carry)
            fetch((ch + 2) % NCHUNK, buf0, sem0)
            wait_fetch(ch + 1, buf1, sem1)
            carry = lax.fori_loop(0, CHUNK_ROWS, process(buf1), carry)
            return carry

        return lax.fori_loop(0, NPAIR, pair_body, carry)

    fetch(0, buf0, sem0)
    big = jnp.full((L,), jnp.inf, jnp.float32)

    def mbody(_buf):
        def body(r, carry):
            mn, mx = carry
            mn, mx = list(mn), list(mx)
            for j in range(VECS_PER_ROW):
                v = _buf[r, pl.ds(j * L, L)]
                k = j % 4
                mn[k] = jnp.minimum(mn[k], v)
                mx[k] = jnp.maximum(mx[k], v)
            return tuple(mn), tuple(mx)
        return body

    mns, mxs = run_pass(mbody, ((big,) * 4, (-big,) * 4))
    mn = jnp.minimum(jnp.minimum(mns[0], mns[1]), jnp.minimum(mns[2], mns[3]))
    mx = jnp.maximum(jnp.maximum(mxs[0], mxs[1]), jnp.maximum(mxs[2], mxs[3]))

    stage[...] = mn
    pltpu.sync_copy(stage, smin.at[s])
    stage[...] = mx
    pltpu.sync_copy(stage, smax.at[s])
    plsc.subcore_barrier()
    pltpu.sync_copy(smin.at[s ^ 1], stage)
    mn = jnp.minimum(mn, stage[...])
    pltpu.sync_copy(smax.at[s ^ 1], stage)
    mx = jnp.maximum(mx, stage[...])

    def _xor_allreduce(vec, op):
        for sh in (1, 2, 4, 8):
            vec = op(vec, _lane_perm(vec, iota ^ sh))
        return vec

    mn = _xor_allreduce(mn, jnp.minimum)
    mx = _xor_allreduce(mx, jnp.maximum)
    scale = (LEVELS - 1.0) / (mx - mn)
    shiftm = MAGIC - mn * scale

    zero = jnp.zeros((L,), jnp.float32)
    for k in range(NACC):
        acc4[k] = zero

    def rbody(_buf):
        def body(r, carry):
            q_prev = _buf[r, pl.ds(0, L)] * scale + shiftm
            for j in range(1, VECS_PER_ROW):
                q = _buf[r, pl.ds(j * L, L)] * scale + shiftm
                u = jnp.where(ge5, q_prev, q)
                d = q_prev - _lane_perm(u, idx5)
                plsc.addupdate(acc4.at[(j - 1) % NACC], d * d)
                q_prev = q
            d = q_prev - _lane_perm(q_prev, idx5)
            plsc.addupdate(acc4.at[3], jnp.where(tail_mask, d * d, 0.0))
            return carry
        return body

    run_pass(rbody, jnp.int32(0))
    wait_fetch(0, buf0, sem0)

    wid = c * NS + s
    res[...] = ((acc4[0] + acc4[1]) + (acc4[2] + acc4[3]))
    pltpu.sync_copy(res, out_hbm.at[pl.ds(wid * L, L)])


_glcm_call = functools.partial(
    pl.kernel,
    out_type=jax.ShapeDtypeStruct((NW * L,), jnp.float32),
    mesh=plsc.VectorSubcoreMesh(core_axis_name="c", subcore_axis_name="s",
                                num_cores=NC, num_subcores=NS),
    compiler_params=pltpu.CompilerParams(needs_layout_passes=False,
                                         use_tc_tiling_on_sc=True),
    scratch_types=[
        pltpu.VMEM((CHUNK_ROWS, W), jnp.float32),
        pltpu.VMEM((CHUNK_ROWS, W), jnp.float32),
        pltpu.VMEM((L,), jnp.float32),
        pltpu.VMEM_SHARED((NS, L), jnp.float32),
        pltpu.VMEM_SHARED((NS, L), jnp.float32),
        pltpu.VMEM((NACC, L), jnp.float32),
        pltpu.VMEM((L,), jnp.float32),
        pltpu.SemaphoreType.DMA,
        pltpu.SemaphoreType.DMA,
    ],
)(_glcm_body)


def kernel(x):
    x2 = x.reshape(B * H, W)
    lane_partials = _glcm_call(x2)
    total = lane_partials.reshape(B, 2 * L).sum(axis=1)
    contrast = total / jnp.float32(NPAIRS)
    return contrast.reshape(B, 1, 1, 1).astype(jnp.float32)


# speedup vs baseline: 2.5471x; 2.5471x over previous
import functools

import jax
import jax.numpy as jnp
from jax import lax
from jax.experimental import pallas as pl
from jax.experimental.pallas import tpu as pltpu
from jax.experimental.pallas import tpu_sc as plsc

LEVELS = 256
OFF = 5
B, H, W = 16, 512, 512
PAIRS_PER_ROW = W - OFF
NPAIRS = H * PAIRS_PER_ROW
NC, NS, L = 2, 16, 16
NW = NC * NS
ROWS_PER_WORKER = H // 2
CHUNK_ROWS = 64
NCHUNK = ROWS_PER_WORKER // CHUNK_ROWS
VECS_PER_ROW = W // L
MAGIC = 12582912.0
_PERM_DN = lax.GatherDimensionNumbers(
    offset_dims=(), collapsed_slice_dims=(0,), start_index_map=(0,))


def _lane_perm(vec, idx):
    return lax.gather(vec, idx[:, None], _PERM_DN, slice_sizes=(1,),
                      mode=lax.GatherScatterMode.PROMISE_IN_BOUNDS)


def _glcm_body(x_hbm, out_hbm, buf0, buf1, stage, smin, smax, res,
               sem0, sem1):
    c = lax.axis_index("c")
    s = lax.axis_index("s")
    img = c * (B // NC) + s // 2
    half = s % 2
    row0 = img * H + half * ROWS_PER_WORKER

    def fetch(ch, buf, sem):
        return pltpu.async_copy(
            x_hbm.at[pl.ds(row0 + ch * CHUNK_ROWS, CHUNK_ROWS), :], buf, sem)

    def wait_fetch(ch, buf, sem):
        pltpu.make_async_copy(
            x_hbm.at[pl.ds(row0 + ch * CHUNK_ROWS, CHUNK_ROWS), :], buf,
            sem).wait()

    iota = lax.iota(jnp.int32, L)
    idx5 = (iota + OFF) & (L - 1)
    ge5 = iota >= OFF
    tail_mask = iota < (PAIRS_PER_ROW - (VECS_PER_ROW - 1) * L)

    NPAIR = NCHUNK // 2

    def run_pass(process, carry):
        def pair_body(p, carry):
            ch = 2 * p
            fetch(ch + 1, buf1, sem1)
            wait_fetch(ch, buf0, sem0)
            carry = lax.fori_loop(0, CHUNK_ROWS, process(buf0), carry)
            fetch((ch + 2) % NCHUNK, buf0, sem0)
            wait_fetch(ch + 1, buf1, sem1)
            carry = lax.fori_loop(0, CHUNK_ROWS, process(buf1), carry)
            return carry

        return lax.fori_loop(0, NPAIR, pair_body, carry)

    fetch(0, buf0, sem0)
    big = jnp.full((L,), jnp.inf, jnp.float32)

    def mbody(_buf):
        def body(r, carry):
            mn, mx = carry
            mn, mx = list(mn), list(mx)
            for j in range(VECS_PER_ROW):
                v = _buf[r, pl.ds(j * L, L)]
                k = j % 4
                mn[k] = jnp.minimum(mn[k], v)
                mx[k] = jnp.maximum(mx[k], v)
            return tuple(mn), tuple(mx)
        return body

    mns, mxs = run_pass(mbody, ((big,) * 4, (-big,) * 4))
    mn = jnp.minimum(jnp.minimum(mns[0], mns[1]), jnp.minimum(mns[2], mns[3]))
    mx = jnp.maximum(jnp.maximum(mxs[0], mxs[1]), jnp.maximum(mxs[2], mxs[3]))

    stage[...] = mn
    pltpu.sync_copy(stage, smin.at[s])
    stage[...] = mx
    pltpu.sync_copy(stage, smax.at[s])
    plsc.subcore_barrier()
    pltpu.sync_copy(smin.at[s ^ 1], stage)
    mn = jnp.minimum(mn, stage[...])
    pltpu.sync_copy(smax.at[s ^ 1], stage)
    mx = jnp.maximum(mx, stage[...])

    def _xor_allreduce(vec, op):
        for sh in (1, 2, 4, 8):
            vec = op(vec, _lane_perm(vec, iota ^ sh))
        return vec

    mn = _xor_allreduce(mn, jnp.minimum)
    mx = _xor_allreduce(mx, jnp.maximum)
    scale = (LEVELS - 1.0) / (mx - mn)
    shiftm = MAGIC - mn * scale

    def rbody(_buf):
        def body(r, accs):
            a0, a1 = accs
            q_prev = _buf[r, pl.ds(0, L)] * scale + shiftm
            for j in range(1, VECS_PER_ROW):
                q = _buf[r, pl.ds(j * L, L)] * scale + shiftm
                u = jnp.where(ge5, q_prev, q)
                d = q_prev - _lane_perm(u, idx5)
                if j % 2:
                    a0 = a0 + d * d
                else:
                    a1 = a1 + d * d
                q_prev = q
            d = q_prev - _lane_perm(q_prev, idx5)
            a1 = a1 + jnp.where(tail_mask, d * d, 0.0)
            return a0, a1
        return body

    zero = jnp.zeros((L,), jnp.float32)
    accs = run_pass(rbody, (zero, zero))
    wait_fetch(0, buf0, sem0)

    wid = c * NS + s
    res[...] = accs[0] + accs[1]
    pltpu.sync_copy(res, out_hbm.at[pl.ds(wid * L, L)])


_glcm_call = functools.partial(
    pl.kernel,
    out_type=jax.ShapeDtypeStruct((NW * L,), jnp.float32),
    mesh=plsc.VectorSubcoreMesh(core_axis_name="c", subcore_axis_name="s",
                                num_cores=NC, num_subcores=NS),
    compiler_params=pltpu.CompilerParams(needs_layout_passes=False,
                                         use_tc_tiling_on_sc=True),
    scratch_types=[
        pltpu.VMEM((CHUNK_ROWS, W), jnp.float32),
        pltpu.VMEM((CHUNK_ROWS, W), jnp.float32),
        pltpu.VMEM((L,), jnp.float32),
        pltpu.VMEM_SHARED((NS, L), jnp.float32),
        pltpu.VMEM_SHARED((NS, L), jnp.float32),
        pltpu.VMEM((L,), jnp.float32),
        pltpu.SemaphoreType.DMA,
        pltpu.SemaphoreType.DMA,
    ],
)(_glcm_body)


def kernel(x):
    x2 = x.reshape(B * H, W)
    lane_partials = _glcm_call(x2)
    total = lane_partials.reshape(B, 2 * L).sum(axis=1)
    contrast = total / jnp.float32(NPAIRS)
    return contrast.reshape(B, 1, 1, 1).astype(jnp.float32)
